# baseline (speedup 1.0000x reference)
import functools

import jax
import jax.numpy as jnp
import numpy as np
from jax import lax
from jax.experimental import pallas as pl
from jax.experimental.pallas import tpu as pltpu
from jax.experimental.pallas import tpu_sc as plsc

N = 320000
D = 128
S = 10000
NC = 2
NS = 16
NW = NC * NS
FW = N // NW
F = 80
NCHUNK = FW // F
PF = 80
NV = D // 16
RPT = 632
CW = 16
DUMMY = S


def _sc_body(frames_hbm, ids_hbm, zsum_hbm, zcnt_hbm,
             sums_out, cnts_out,
             fbuf, idbuf, pbuf, pidx, pcnt, ssum, scnt, gsem):
    cid = lax.axis_index("c")
    sid = lax.axis_index("s")
    wid = cid * NS + sid
    r0 = jnp.minimum(sid * RPT, S - RPT)

    fbase = wid * FW
    ibase = wid * NCHUNK

    def issue_gather(k, bb):
        pltpu.async_copy(frames_hbm.at[pl.ds(fbase + k * F, F)],
                         fbuf.at[pl.ds(bb * F, F)], gsem.at[bb])
        pltpu.async_copy(ids_hbm.at[pl.ds(ibase + k, 1)],
                         idbuf.at[pl.ds(bb, 1)], gsem.at[bb])

    def wait_gather(bb):
        pltpu.make_async_copy(frames_hbm.at[pl.ds(0, F)],
                              fbuf.at[pl.ds(bb * F, F)], gsem.at[bb]).wait()
        pltpu.make_async_copy(ids_hbm.at[pl.ds(0, 1)],
                              idbuf.at[pl.ds(bb, 1)], gsem.at[bb]).wait()

    issue_gather(0, 0)
    issue_gather(1, 1)

    pltpu.sync_copy(zsum_hbm.at[pl.ds(r0, RPT)], ssum.at[pl.ds(r0, RPT)])
    pltpu.sync_copy(zcnt_hbm.at[pl.ds(r0, RPT)], scnt.at[pl.ds(r0, RPT)])
    plsc.subcore_barrier()

    zvec = jnp.zeros((16,), jnp.float32)
    lane_iota = lax.iota(jnp.int32, 16)
    for p in range(PF):
        pcnt[p, :] = zvec

    def flush_scatter():
        pltpu.sync_copy(pbuf, ssum.at[pidx.at[0]], add=True)
        pltpu.sync_copy(pcnt, scnt.at[pidx.at[0]], add=True)

    def splat(x):
        return jnp.full((16,), x, jnp.int32)

    def append_store(acc, cur_id, runlen, nf):
        seg = jnp.where(runlen > 0, cur_id, DUMMY)
        al = (nf // 16) * 16
        cur = pidx[0, pl.ds(al, 16)]
        ind = jnp.maximum(0, 1 - jnp.abs(lane_iota - splat(nf - al)))
        pidx.at[0][pl.ds(al, 16)] = cur + (splat(seg) - cur) * ind
        for v in range(NV):
            pbuf.at[nf][pl.ds(v * 16, 16)] = acc[v]
        cntv = jnp.full((16,), runlen.astype(jnp.float32), jnp.float32)
        pcnt.at[nf][pl.ds(0, 16)] = cntv

    def append_run(acc, cur_id, runlen, nf):
        append_store(acc, cur_id, runlen, nf)
        nf = nf + 1
        flush_now = nf == PF

        @pl.when(flush_now)
        def _():
            flush_scatter()

        return jnp.where(flush_now, jnp.int32(0), nf)

    def group_body(g, carry):
        (*acc, cur_id, runlen, nf, bb) = carry
        idvec = idbuf[bb, pl.ds(g * 16, 16)]
        base = bb * F + g * 16

        for lane in range(16):
            fid = idvec[lane]
            row = [fbuf[base + lane, pl.ds(v * 16, 16)] for v in range(NV)]
            b = fid != cur_id

            @pl.when(b)
            def _(acc=acc, cur_id=cur_id, runlen=runlen, nf=nf):
                append_store(acc, cur_id, runlen, nf)

            nf = jnp.where(b, nf + 1, nf)
            flush_now = nf == PF

            @pl.when(flush_now)
            def _():
                flush_scatter()

            nf = jnp.where(flush_now, jnp.int32(0), nf)
            keep = jnp.full((16,), jnp.where(b, 0.0, 1.0), jnp.float32)
            acc = [r + a * keep for a, r in zip(acc, row)]
            runlen = jnp.where(b, jnp.int32(1), runlen + 1)
            cur_id = fid
        return (*acc, cur_id, runlen, nf, bb)

    def chunk_body(k, carry):
        bb = lax.rem(k, 2)
        wait_gather(bb)
        carry = lax.fori_loop(0, F // 16, group_body, (*carry, bb))[:-1]

        @pl.when(k + 2 < NCHUNK)
        def _():
            issue_gather(k + 2, bb)

        return carry

    acc0 = [zvec] * NV
    init = (*acc0, jnp.int32(-1), jnp.int32(0), jnp.int32(0))
    (*acc, cur_id, runlen, nf) = lax.fori_loop(0, NCHUNK, chunk_body, init)

    nf = append_run(acc, cur_id, runlen, nf)

    dummyv = jnp.full((16,), DUMMY, jnp.int32)
    for g in range(PF // 16):
        cur = pidx[0, pl.ds(g * 16, 16)]
        d = lane_iota + g * 16 - splat(nf)
        tail = jnp.minimum(1, jnp.maximum(0, d + 1))
        pidx.at[0][pl.ds(g * 16, 16)] = cur + (dummyv - cur) * tail

    @pl.when(nf > 0)
    def _():
        flush_scatter()

    plsc.subcore_barrier()

    pltpu.async_copy(ssum.at[pl.ds(r0, RPT)],
                     sums_out.at[pl.ds(cid * S + r0, RPT)], gsem.at[0])
    pltpu.async_copy(scnt.at[pl.ds(r0, RPT)],
                     cnts_out.at[pl.ds(cid * S + r0, RPT)], gsem.at[1])
    pltpu.make_async_copy(ssum.at[pl.ds(r0, RPT)],
                          sums_out.at[pl.ds(cid * S + r0, RPT)], gsem.at[0]).wait()
    pltpu.make_async_copy(scnt.at[pl.ds(r0, RPT)],
                          cnts_out.at[pl.ds(cid * S + r0, RPT)], gsem.at[1]).wait()


_sc_segment_sum = functools.partial(
    pl.kernel,
    out_type=[
        jax.ShapeDtypeStruct((NC * S, D), jnp.float32),
        jax.ShapeDtypeStruct((NC * S, CW), jnp.float32),
    ],
    mesh=plsc.VectorSubcoreMesh(core_axis_name="c", subcore_axis_name="s"),
    compiler_params=pltpu.CompilerParams(use_tc_tiling_on_sc=False),
    scratch_types=[
        pltpu.VMEM((2 * F, D), jnp.float32),
        pltpu.VMEM((2, F), jnp.int32),
        pltpu.VMEM((PF, D), jnp.float32),
        pltpu.VMEM((1, PF), jnp.int32),
        pltpu.VMEM((PF, CW), jnp.float32),
        pltpu.VMEM_SHARED((S + 8, D), jnp.float32),
        pltpu.VMEM_SHARED((S + 8, CW), jnp.float32),
        pltpu.SemaphoreType.DMA((2,)),
    ],
)(_sc_body)


_BS = 1000


def _combine_body(s_ref, c_ref, o_ref):
    s = s_ref[0] + s_ref[1]
    c = c_ref[0, :, 0:1] + c_ref[1, :, 0:1]
    o_ref[...] = s / (c + 1e-8)


_combine = pl.pallas_call(
    _combine_body,
    grid=(S // _BS,),
    in_specs=[
        pl.BlockSpec((2, _BS, D), lambda i: (0, i, 0)),
        pl.BlockSpec((2, _BS, CW), lambda i: (0, i, 0)),
    ],
    out_specs=pl.BlockSpec((_BS, D), lambda i: (i, 0)),
    out_shape=jax.ShapeDtypeStruct((S, D), jnp.float32),
)


_ZSUM = np.zeros((S, D), np.float32)
_ZCNT = np.zeros((S, CW), np.float32)


def kernel(frame_features, segment_ids, num_segments):
    ids2d = segment_ids.astype(jnp.int32).reshape(N // F, F)
    sums, cnts = _sc_segment_sum(frame_features, ids2d, _ZSUM, _ZCNT)
    return _combine(sums.reshape(NC, S, D), cnts.reshape(NC, S, CW))


# speedup vs baseline: 1.5211x; 1.5211x over previous
import functools

import jax
import jax.numpy as jnp
import numpy as np
from jax import lax
from jax.experimental import pallas as pl
from jax.experimental.pallas import tpu as pltpu
from jax.experimental.pallas import tpu_sc as plsc

N = 320000
D = 128
S = 10000
NC = 2
NS = 16
NW = NC * NS
FW = N // NW
F = 80
NCHUNK = FW // F
NSUP = 41
PF = 48
NV = D // 16
RPT = 632
CW = 16
DUMMY = S


def _sc_body(frames_hbm, ids_hbm, zsum_hbm, zcnt_hbm, ones_hbm,
             sums_out, cnts_out,
             fbuf, idbuf, ones_v, pbuf, pidx, pcnt, ssum, scnt, gsem, ssem):
    cid = lax.axis_index("c")
    sid = lax.axis_index("s")
    wid = cid * NS + sid
    r0 = jnp.minimum(sid * RPT, S - RPT)

    fbase = wid * FW
    ibase = wid * NCHUNK

    def issue_gather(k, bb):
        pltpu.async_copy(frames_hbm.at[pl.ds(fbase + k * F, F)],
                         fbuf.at[pl.ds(bb * F, F)], gsem.at[bb])
        pltpu.async_copy(ids_hbm.at[pl.ds(ibase + k, 1)],
                         idbuf.at[pl.ds(bb, 1)], gsem.at[bb])

    def wait_gather(bb):
        pltpu.make_async_copy(frames_hbm.at[pl.ds(0, F)],
                              fbuf.at[pl.ds(bb * F, F)], gsem.at[bb]).wait()
        pltpu.make_async_copy(ids_hbm.at[pl.ds(0, 1)],
                              idbuf.at[pl.ds(bb, 1)], gsem.at[bb]).wait()

    def issue_stream_scatters(bb):
        row = idbuf.at[bb]
        pltpu.async_copy(fbuf.at[pl.ds(bb * F, F)], ssum.at[row],
                         ssem.at[bb], add=True)
        pltpu.async_copy(ones_v, scnt.at[row], ssem.at[bb], add=True)

    def wait_stream_scatters(bb):
        row = idbuf.at[bb]
        pltpu.make_async_copy(fbuf.at[pl.ds(bb * F, F)], ssum.at[row],
                              ssem.at[bb]).wait()
        pltpu.make_async_copy(ones_v, scnt.at[row], ssem.at[bb]).wait()

    issue_gather(0, 0)
    issue_gather(1, 1)
    issue_gather(2, 2)

    pltpu.sync_copy(zsum_hbm.at[pl.ds(r0, RPT)], ssum.at[pl.ds(r0, RPT)])
    pltpu.sync_copy(zcnt_hbm.at[pl.ds(r0, RPT)], scnt.at[pl.ds(r0, RPT)])
    pltpu.sync_copy(ones_hbm, ones_v)
    plsc.subcore_barrier()

    zvec = jnp.zeros((16,), jnp.float32)
    lane_iota = lax.iota(jnp.int32, 16)
    for p in range(PF):
        pcnt[p, :] = zvec

    def splat(x):
        return jnp.full((16,), x, jnp.int32)

    def flush_scatter():
        pltpu.sync_copy(pbuf, ssum.at[pidx.at[0]], add=True)
        pltpu.sync_copy(pcnt, scnt.at[pidx.at[0]], add=True)

    def append_store(acc, cur_id, runlen, nf):
        seg = jnp.where(runlen > 0, cur_id, DUMMY)
        al = (nf // 16) * 16
        cur = pidx[0, pl.ds(al, 16)]
        ind = jnp.maximum(0, 1 - jnp.abs(lane_iota - splat(nf - al)))
        pidx.at[0][pl.ds(al, 16)] = cur + (splat(seg) - cur) * ind
        for v in range(NV):
            pbuf.at[nf][pl.ds(v * 16, 16)] = acc[v]
        cntv = jnp.full((16,), runlen.astype(jnp.float32), jnp.float32)
        pcnt.at[nf][pl.ds(0, 16)] = cntv

    def group_body(g, carry):
        (*acc, cur_id, runlen, nf, bb) = carry
        idvec = idbuf[bb, pl.ds(g * 16, 16)]
        base = bb * F + g * 16

        for lane in range(16):
            fid = idvec[lane]
            row = [fbuf[base + lane, pl.ds(v * 16, 16)] for v in range(NV)]
            b = fid != cur_id

            @pl.when(b)
            def _(acc=acc, cur_id=cur_id, runlen=runlen, nf=nf):
                append_store(acc, cur_id, runlen, nf)

            nf = jnp.where(b, nf + 1, nf)
            flush_now = nf == PF

            @pl.when(flush_now)
            def _():
                flush_scatter()

            nf = jnp.where(flush_now, jnp.int32(0), nf)
            keep = jnp.full((16,), jnp.where(b, 0.0, 1.0), jnp.float32)
            acc = [r + a * keep for a, r in zip(acc, row)]
            runlen = jnp.where(b, jnp.int32(1), runlen + 1)
            cur_id = fid
        return (*acc, cur_id, runlen, nf, bb)

    def super_body(s, carry):
        for bb in range(2):
            wait_gather(bb)
            issue_stream_scatters(bb)

        wait_gather(2)
        carry = lax.fori_loop(0, F // 16, group_body, (*carry, jnp.int32(2)))[:-1]

        for bb in range(2):
            wait_stream_scatters(bb)

            @pl.when(3 * s + 3 + bb < NCHUNK)
            def _(bb=bb):
                issue_gather(3 * s + 3 + bb, bb)

        @pl.when(3 * s + 5 < NCHUNK)
        def _():
            issue_gather(3 * s + 5, 2)

        return carry

    acc0 = [zvec] * NV
    init = (*acc0, jnp.int32(-1), jnp.int32(0), jnp.int32(0))
    carry = lax.fori_loop(0, NSUP, super_body, init)

    for bb in range(2):
        wait_gather(bb)
        issue_stream_scatters(bb)

    (*acc, cur_id, runlen, nf) = carry
    append_store(acc, cur_id, runlen, nf)
    nf = nf + 1

    dummyv = jnp.full((16,), DUMMY, jnp.int32)
    for g in range(PF // 16):
        cur = pidx[0, pl.ds(g * 16, 16)]
        d = lane_iota + g * 16 - splat(nf)
        tail = jnp.minimum(1, jnp.maximum(0, d + 1))
        pidx.at[0][pl.ds(g * 16, 16)] = cur + (dummyv - cur) * tail

    flush_scatter()

    for bb in range(2):
        wait_stream_scatters(bb)
    plsc.subcore_barrier()

    pltpu.async_copy(ssum.at[pl.ds(r0, RPT)],
                     sums_out.at[pl.ds(cid * S + r0, RPT)], gsem.at[0])
    pltpu.async_copy(scnt.at[pl.ds(r0, RPT)],
                     cnts_out.at[pl.ds(cid * S + r0, RPT)], gsem.at[1])
    pltpu.make_async_copy(ssum.at[pl.ds(r0, RPT)],
                          sums_out.at[pl.ds(cid * S + r0, RPT)], gsem.at[0]).wait()
    pltpu.make_async_copy(scnt.at[pl.ds(r0, RPT)],
                          cnts_out.at[pl.ds(cid * S + r0, RPT)], gsem.at[1]).wait()


_sc_segment_sum = functools.partial(
    pl.kernel,
    out_type=[
        jax.ShapeDtypeStruct((NC * S, D), jnp.float32),
        jax.ShapeDtypeStruct((NC * S, CW), jnp.float32),
    ],
    mesh=plsc.VectorSubcoreMesh(core_axis_name="c", subcore_axis_name="s"),
    compiler_params=pltpu.CompilerParams(use_tc_tiling_on_sc=False),
    scratch_types=[
        pltpu.VMEM((3 * F, D), jnp.float32),
        pltpu.VMEM((3, F), jnp.int32),
        pltpu.VMEM((F, CW), jnp.float32),
        pltpu.VMEM((PF, D), jnp.float32),
        pltpu.VMEM((1, PF), jnp.int32),
        pltpu.VMEM((PF, CW), jnp.float32),
        pltpu.VMEM_SHARED((S + 8, D), jnp.float32),
        pltpu.VMEM_SHARED((S + 8, CW), jnp.float32),
        pltpu.SemaphoreType.DMA((3,)),
        pltpu.SemaphoreType.DMA((2,)),
    ],
)(_sc_body)


_BS = 1000


def _combine_body(s_ref, c_ref, o_ref):
    s = s_ref[0] + s_ref[1]
    c = c_ref[0, :, 0:1] + c_ref[1, :, 0:1]
    o_ref[...] = s / (c + 1e-8)


_combine = pl.pallas_call(
    _combine_body,
    grid=(S // _BS,),
    in_specs=[
        pl.BlockSpec((2, _BS, D), lambda i: (0, i, 0)),
        pl.BlockSpec((2, _BS, CW), lambda i: (0, i, 0)),
    ],
    out_specs=pl.BlockSpec((_BS, D), lambda i: (i, 0)),
    out_shape=jax.ShapeDtypeStruct((S, D), jnp.float32),
)


_ZSUM = np.zeros((S, D), np.float32)
_ZCNT = np.zeros((S, CW), np.float32)
_ONES = np.ones((F, CW), np.float32)


def kernel(frame_features, segment_ids, num_segments):
    ids2d = segment_ids.astype(jnp.int32).reshape(N // F, F)
    sums, cnts = _sc_segment_sum(frame_features, ids2d, _ZSUM, _ZCNT, _ONES)
    return _combine(sums.reshape(NC, S, D), cnts.reshape(NC, S, CW))


# speedup vs baseline: 2.3838x; 1.5671x over previous
import functools

import jax
import jax.numpy as jnp
import numpy as np
from jax import lax
from jax.experimental import pallas as pl
from jax.experimental.pallas import tpu as pltpu
from jax.experimental.pallas import tpu_sc as plsc

N = 320000
D = 128
S = 10000
NC = 2
NS = 16
NW = NC * NS
FW = N // NW
F = 80
NCHUNK = FW // F
NBUF = 3
RPT = 632
CW = 16


def _sc_body(frames_hbm, ids_hbm, zsum_hbm, zcnt_hbm, ones_hbm,
             sums_out, cnts_out,
             fbuf, idbuf, ones_v, ssum, scnt, gsem, ssem):
    cid = lax.axis_index("c")
    sid = lax.axis_index("s")
    wid = cid * NS + sid
    r0 = jnp.minimum(sid * RPT, S - RPT)

    fbase = wid * FW
    ibase = wid * NCHUNK

    def issue_gather(k, bb):
        pltpu.async_copy(frames_hbm.at[pl.ds(fbase + k * F, F)],
                         fbuf.at[pl.ds(bb * F, F)], gsem.at[bb])
        pltpu.async_copy(ids_hbm.at[pl.ds(ibase + k, 1)],
                         idbuf.at[pl.ds(bb, 1)], gsem.at[bb])

    def wait_gather(bb):
        pltpu.make_async_copy(frames_hbm.at[pl.ds(0, F)],
                              fbuf.at[pl.ds(bb * F, F)], gsem.at[bb]).wait()
        pltpu.make_async_copy(ids_hbm.at[pl.ds(0, 1)],
                              idbuf.at[pl.ds(bb, 1)], gsem.at[bb]).wait()

    def issue_scatters(bb):
        row = idbuf.at[bb]
        pltpu.async_copy(fbuf.at[pl.ds(bb * F, F)], ssum.at[row],
                         ssem.at[bb], add=True)
        pltpu.async_copy(ones_v, scnt.at[row], ssem.at[bb], add=True)

    def wait_scatters(bb):
        row = idbuf.at[bb]
        pltpu.make_async_copy(fbuf.at[pl.ds(bb * F, F)], ssum.at[row],
                              ssem.at[bb]).wait()
        pltpu.make_async_copy(ones_v, scnt.at[row], ssem.at[bb]).wait()

    issue_gather(0, 0)
    issue_gather(1, 1)

    pltpu.sync_copy(zsum_hbm.at[pl.ds(r0, RPT)], ssum.at[pl.ds(r0, RPT)])
    pltpu.sync_copy(zcnt_hbm.at[pl.ds(r0, RPT)], scnt.at[pl.ds(r0, RPT)])
    pltpu.sync_copy(ones_hbm, ones_v)
    plsc.subcore_barrier()

    def body(k, carry):
        bb = lax.rem(k, NBUF)
        nb = lax.rem(k + 2, NBUF)
        wait_gather(bb)

        @pl.when(k >= 1)
        def _():
            wait_scatters(nb)

        @pl.when(k + 2 < NCHUNK)
        def _():
            issue_gather(k + 2, nb)

        issue_scatters(bb)
        return carry

    lax.fori_loop(0, NCHUNK, body, 0)
    wait_scatters((NCHUNK - 1) % NBUF)
    plsc.subcore_barrier()

    pltpu.async_copy(ssum.at[pl.ds(r0, RPT)],
                     sums_out.at[pl.ds(cid * S + r0, RPT)], gsem.at[0])
    pltpu.async_copy(scnt.at[pl.ds(r0, RPT)],
                     cnts_out.at[pl.ds(cid * S + r0, RPT)], gsem.at[1])
    pltpu.make_async_copy(ssum.at[pl.ds(r0, RPT)],
                          sums_out.at[pl.ds(cid * S + r0, RPT)], gsem.at[0]).wait()
    pltpu.make_async_copy(scnt.at[pl.ds(r0, RPT)],
                          cnts_out.at[pl.ds(cid * S + r0, RPT)], gsem.at[1]).wait()


_sc_segment_sum = functools.partial(
    pl.kernel,
    out_type=[
        jax.ShapeDtypeStruct((NC * S, D), jnp.float32),
        jax.ShapeDtypeStruct((NC * S, CW), jnp.float32),
    ],
    mesh=plsc.VectorSubcoreMesh(core_axis_name="c", subcore_axis_name="s"),
    compiler_params=pltpu.CompilerParams(use_tc_tiling_on_sc=False),
    scratch_types=[
        pltpu.VMEM((NBUF * F, D), jnp.float32),
        pltpu.VMEM((NBUF, F), jnp.int32),
        pltpu.VMEM((F, CW), jnp.float32),
        pltpu.VMEM_SHARED((S, D), jnp.float32),
        pltpu.VMEM_SHARED((S, CW), jnp.float32),
        pltpu.SemaphoreType.DMA((NBUF,)),
        pltpu.SemaphoreType.DMA((NBUF,)),
    ],
)(_sc_body)


_BS = 1000


def _combine_body(s_ref, c_ref, o_ref):
    s = s_ref[0] + s_ref[1]
    c = c_ref[0, :, 0:1] + c_ref[1, :, 0:1]
    o_ref[...] = s / (c + 1e-8)


_combine = pl.pallas_call(
    _combine_body,
    grid=(S // _BS,),
    in_specs=[
        pl.BlockSpec((2, _BS, D), lambda i: (0, i, 0)),
        pl.BlockSpec((2, _BS, CW), lambda i: (0, i, 0)),
    ],
    out_specs=pl.BlockSpec((_BS, D), lambda i: (i, 0)),
    out_shape=jax.ShapeDtypeStruct((S, D), jnp.float32),
)


_ZSUM = np.zeros((S, D), np.float32)
_ZCNT = np.zeros((S, CW), np.float32)
_ONES = np.ones((F, CW), np.float32)


def kernel(frame_features, segment_ids, num_segments):
    ids2d = segment_ids.astype(jnp.int32).reshape(N // F, F)
    sums, cnts = _sc_segment_sum(frame_features, ids2d, _ZSUM, _ZCNT, _ONES)
    return _combine(sums.reshape(NC, S, D), cnts.reshape(NC, S, CW))
